# baseline (speedup 1.0000x reference)
import functools

import jax
import jax.numpy as jnp
from jax import lax
from jax.experimental import pallas as pl
from jax.experimental.pallas import tpu as pltpu
from jax.experimental.pallas import tpu_sc as plsc

B = 16384
D = 64
DC = D // 16
NC = 2
NS = 16
NW = NC * NS
PER_W = B // NW
CHUNK = 128
NCHUNK = PER_W // CHUNK
GROUPS = CHUNK // 16
PAD = 17

_MESH = plsc.VectorSubcoreMesh(core_axis_name="c", subcore_axis_name="s")

_BCAST_DNUMS = lax.GatherDimensionNumbers(
    offset_dims=(), collapsed_slice_dims=(0,), start_index_map=(0,))


def _lane_broadcast(vec, j):
    idx = jnp.full((16, 1), j, jnp.int32)
    return lax.gather(vec, idx, _BCAST_DNUMS, (1,),
                      mode=lax.GatherScatterMode.PROMISE_IN_BOUNDS)


_CONCAT_BLK = 1024


def _concat_body(a_ref, b_ref, o_ref):
    a2 = a_ref[...].reshape(_CONCAT_BLK // 2, 2 * D)
    b2 = b_ref[...].reshape(_CONCAT_BLK // 2, 2 * D)
    left = jnp.concatenate([a2[:, :D], b2[:, :D]], axis=1)
    right = jnp.concatenate([a2[:, D:], b2[:, D:]], axis=1)
    o_ref[...] = jnp.stack([left, right], axis=1).reshape(_CONCAT_BLK, 2 * D)


def _tc_concat(a, b, n):
    grid = (n + _CONCAT_BLK - 1) // _CONCAT_BLK
    return pl.pallas_call(
        _concat_body,
        grid=(grid,),
        in_specs=[
            pl.BlockSpec((_CONCAT_BLK * D,), lambda i: (i,)),
            pl.BlockSpec((_CONCAT_BLK * D,), lambda i: (i,)),
        ],
        out_specs=pl.BlockSpec((_CONCAT_BLK, 2 * D), lambda i: (i, 0)),
        out_shape=jax.ShapeDtypeStruct((n, 2 * D), jnp.float32),
    )(a.reshape(-1), b.reshape(-1))


@functools.partial(
    pl.kernel,
    mesh=_MESH,
    compiler_params=pltpu.CompilerParams(
        needs_layout_passes=False, use_tc_tiling_on_sc=True),
    out_type=[
        jax.ShapeDtypeStruct((B,), jnp.float32),
        jax.ShapeDtypeStruct((B * D,), jnp.float32),
        jax.ShapeDtypeStruct((B * D,), jnp.float32),
    ],
    scratch_types=[
        pltpu.VMEM((CHUNK,), jnp.int32),
        pltpu.VMEM((CHUNK,), jnp.int32),
        pltpu.VMEM((CHUNK,), jnp.int32),
        pltpu.VMEM((CHUNK, 2 * D), jnp.float32),
        pltpu.VMEM((CHUNK, 2 * D), jnp.float32),
        pltpu.VMEM((CHUNK, 2 * D), jnp.float32),
        pltpu.VMEM((CHUNK * PAD,), jnp.float32),
        pltpu.VMEM((CHUNK * PAD,), jnp.float32),
        pltpu.VMEM((CHUNK * PAD,), jnp.float32),
        pltpu.VMEM((CHUNK * D,), jnp.float32),
        pltpu.VMEM((CHUNK * D,), jnp.float32),
        pltpu.VMEM((CHUNK,), jnp.float32),
        pltpu.SemaphoreType.DMA,
    ],
)
def _sc_transd(h_hbm, t_hbm, r_hbm, entc_hbm, relc_hbm,
               score_hbm, ph_hbm, pt_hbm,
               h_idx, t_idx, r_idx, h_b, t_b, r_b,
               hd_part, td_part, sc_part,
               ph_v, pt_v, sc_v, sem):
    wid = lax.axis_index("s") * NC + lax.axis_index("c")
    base = wid * PER_W
    iota16 = lax.iota(jnp.int32, 16)
    row17 = iota16 * PAD

    def chunk_body(c, carry):
        off = base + c * CHUNK
        pltpu.sync_copy(h_hbm.at[pl.ds(off, CHUNK)], h_idx)
        pltpu.sync_copy(t_hbm.at[pl.ds(off, CHUNK)], t_idx)
        pltpu.sync_copy(r_hbm.at[pl.ds(off, CHUNK)], r_idx)
        copies = [
            pltpu.async_copy(entc_hbm.at[h_idx], h_b, sem),
            pltpu.async_copy(entc_hbm.at[t_idx], t_b, sem),
            pltpu.async_copy(relc_hbm.at[r_idx], r_b, sem),
        ]
        for cp in copies:
            cp.wait()

        def dot_body(i, dcarry):
            hd = h_b[i, pl.ds(0, 16)] * h_b[i, pl.ds(D, 16)]
            td = t_b[i, pl.ds(0, 16)] * t_b[i, pl.ds(D, 16)]
            for dc in range(1, DC):
                sl = pl.ds(dc * 16, 16)
                slp = pl.ds(D + dc * 16, 16)
                hd = hd + h_b[i, sl] * h_b[i, slp]
                td = td + t_b[i, sl] * t_b[i, slp]
            hd_part[pl.ds(i * PAD, 16)] = hd
            td_part[pl.ds(i * PAD, 16)] = td
            return dcarry

        lax.fori_loop(0, CHUNK, dot_body, 0)

        def group_body(g, gcarry):
            grow = g * (16 * PAD) + row17
            sh = plsc.load_gather(hd_part, [grow])
            st = plsc.load_gather(td_part, [grow])
            for l in range(1, 16):
                gl = grow + l
                sh = sh + plsc.load_gather(hd_part, [gl])
                st = st + plsc.load_gather(td_part, [gl])
            for j in range(16):
                i = g * 16 + j
                shi = _lane_broadcast(sh, j)
                sti = _lane_broadcast(st, j)
                sl0 = pl.ds(0, 16)
                slp0 = pl.ds(D, 16)
                rp = r_b[i, slp0]
                ph = h_b[i, sl0] + shi * rp
                pt = t_b[i, sl0] + sti * rp
                ph_v[pl.ds(i * D, 16)] = ph
                pt_v[pl.ds(i * D, 16)] = pt
                diff = ph + r_b[i, sl0] - pt
                acc = diff * diff
                for dc in range(1, DC):
                    sl = pl.ds(dc * 16, 16)
                    slp = pl.ds(D + dc * 16, 16)
                    rp = r_b[i, slp]
                    ph = h_b[i, sl] + shi * rp
                    pt = t_b[i, sl] + sti * rp
                    ph_v[pl.ds(i * D + dc * 16, 16)] = ph
                    pt_v[pl.ds(i * D + dc * 16, 16)] = pt
                    diff = ph + r_b[i, sl] - pt
                    acc = acc + diff * diff
                sc_part[pl.ds(i * PAD, 16)] = acc
            sc = plsc.load_gather(sc_part, [grow])
            for l in range(1, 16):
                sc = sc + plsc.load_gather(sc_part, [grow + l])
            sc_v[pl.ds(g * 16, 16)] = sc
            return gcarry

        lax.fori_loop(0, GROUPS, group_body, 0)
        pltpu.sync_copy(ph_v, ph_hbm.at[pl.ds(off * D, CHUNK * D)])
        pltpu.sync_copy(pt_v, pt_hbm.at[pl.ds(off * D, CHUNK * D)])
        pltpu.sync_copy(sc_v, score_hbm.at[pl.ds(off, CHUNK)])
        return carry

    lax.fori_loop(0, NCHUNK, chunk_body, 0)


def kernel(ratings, triples, ent_emb, rel_emb, ent_proj_emb, rel_proj_emb):
    h = triples[0]
    t = triples[1]
    r = triples[2]
    entc = _tc_concat(ent_emb, ent_proj_emb, ent_emb.shape[0])
    relc = _tc_concat(rel_emb, rel_proj_emb, rel_emb.shape[0])
    score, ph_flat, pt_flat = _sc_transd(h, t, r, entc, relc)
    proj_h_e = ph_flat.reshape(B, D)
    proj_t_e = pt_flat.reshape(B, D)
    ones = jnp.ones((512, 64), dtype=jnp.float32)
    return (score, proj_h_e, proj_t_e, ones, ones)


# speedup vs baseline: 1.8735x; 1.8735x over previous
import functools

import jax
import jax.numpy as jnp
from jax import lax
from jax.experimental import pallas as pl
from jax.experimental.pallas import tpu as pltpu
from jax.experimental.pallas import tpu_sc as plsc

B = 16384
D = 64
DC = D // 16
NC = 2
NS = 16
NW = NC * NS
PER_W = B // NW
CHUNK = 128
NCHUNK = PER_W // CHUNK
GROUPS = CHUNK // 16
PAD = 17

_MESH = plsc.VectorSubcoreMesh(core_axis_name="c", subcore_axis_name="s")

_BCAST_DNUMS = lax.GatherDimensionNumbers(
    offset_dims=(), collapsed_slice_dims=(0,), start_index_map=(0,))


def _lane_broadcast(vec, j):
    idx = jnp.full((16, 1), j, jnp.int32)
    return lax.gather(vec, idx, _BCAST_DNUMS, (1,),
                      mode=lax.GatherScatterMode.PROMISE_IN_BOUNDS)


@functools.partial(
    pl.kernel,
    mesh=_MESH,
    compiler_params=pltpu.CompilerParams(
        needs_layout_passes=False, use_tc_tiling_on_sc=True),
    out_type=[
        jax.ShapeDtypeStruct((B,), jnp.float32),
        jax.ShapeDtypeStruct((B * D,), jnp.float32),
        jax.ShapeDtypeStruct((B * D,), jnp.float32),
    ],
    scratch_types=[
        pltpu.VMEM((PER_W,), jnp.int32),
        pltpu.VMEM((PER_W,), jnp.int32),
        pltpu.VMEM((PER_W,), jnp.int32),
        pltpu.VMEM((CHUNK, 2 * D), jnp.float32),
        pltpu.VMEM((CHUNK, 2 * D), jnp.float32),
        pltpu.VMEM((CHUNK, 2 * D), jnp.float32),
        pltpu.VMEM((CHUNK, 2 * D), jnp.float32),
        pltpu.VMEM((CHUNK, 2 * D), jnp.float32),
        pltpu.VMEM((CHUNK, 2 * D), jnp.float32),
        pltpu.VMEM((CHUNK * PAD,), jnp.float32),
        pltpu.VMEM((CHUNK * PAD,), jnp.float32),
        pltpu.VMEM((CHUNK * PAD,), jnp.float32),
        pltpu.VMEM((CHUNK * D,), jnp.float32),
        pltpu.VMEM((CHUNK * D,), jnp.float32),
        pltpu.VMEM((CHUNK,), jnp.float32),
        pltpu.SemaphoreType.DMA,
        pltpu.SemaphoreType.DMA,
    ],
)
def _sc_transd(h_hbm, t_hbm, r_hbm, entc_hbm, relc_hbm,
               score_hbm, ph_hbm, pt_hbm,
               h_idx, t_idx, r_idx,
               h_a, t_a, r_a, h_b2, t_b2, r_b2,
               hd_part, td_part, sc_part,
               ph_v, pt_v, sc_v, sem_a, sem_b):
    wid = lax.axis_index("s") * NC + lax.axis_index("c")
    base = wid * PER_W
    iota16 = lax.iota(jnp.int32, 16)
    row17 = iota16 * PAD

    pltpu.sync_copy(h_hbm.at[pl.ds(base, PER_W)], h_idx)
    pltpu.sync_copy(t_hbm.at[pl.ds(base, PER_W)], t_idx)
    pltpu.sync_copy(r_hbm.at[pl.ds(base, PER_W)], r_idx)

    def gathers(c, hb, tb, rb, sem):
        sl = pl.ds(c * CHUNK, CHUNK)
        return [
            pltpu.async_copy(entc_hbm.at[h_idx.at[sl]], hb, sem),
            pltpu.async_copy(entc_hbm.at[t_idx.at[sl]], tb, sem),
            pltpu.async_copy(relc_hbm.at[r_idx.at[sl]], rb, sem),
        ]

    def fire(c, hb, tb, rb, sem):
        for cp in gathers(c, hb, tb, rb, sem):
            pass

    def drain(c, hb, tb, rb, sem):
        for cp in [pltpu.make_async_copy(entc_hbm.at[h_idx.at[pl.ds(0, CHUNK)]], hb, sem),
                   pltpu.make_async_copy(entc_hbm.at[t_idx.at[pl.ds(0, CHUNK)]], tb, sem),
                   pltpu.make_async_copy(relc_hbm.at[r_idx.at[pl.ds(0, CHUNK)]], rb, sem)]:
            cp.wait()

    def compute(c, h_b, t_b, r_b):
        off = base + c * CHUNK

        def dot_body(i, dcarry):
            hd = h_b[i, pl.ds(0, 16)] * h_b[i, pl.ds(D, 16)]
            td = t_b[i, pl.ds(0, 16)] * t_b[i, pl.ds(D, 16)]
            for dc in range(1, DC):
                sl = pl.ds(dc * 16, 16)
                slp = pl.ds(D + dc * 16, 16)
                hd = hd + h_b[i, sl] * h_b[i, slp]
                td = td + t_b[i, sl] * t_b[i, slp]
            hd_part[pl.ds(i * PAD, 16)] = hd
            td_part[pl.ds(i * PAD, 16)] = td
            return dcarry

        lax.fori_loop(0, CHUNK, dot_body, 0)

        def group_body(g, gcarry):
            grow = g * (16 * PAD) + row17
            sh = plsc.load_gather(hd_part, [grow])
            st = plsc.load_gather(td_part, [grow])
            for l in range(1, 16):
                gl = grow + l
                sh = sh + plsc.load_gather(hd_part, [gl])
                st = st + plsc.load_gather(td_part, [gl])
            for j in range(16):
                i = g * 16 + j
                shi = _lane_broadcast(sh, j)
                sti = _lane_broadcast(st, j)
                acc = None
                for dc in range(DC):
                    sl = pl.ds(dc * 16, 16)
                    slp = pl.ds(D + dc * 16, 16)
                    rp = r_b[i, slp]
                    ph = h_b[i, sl] + shi * rp
                    pt = t_b[i, sl] + sti * rp
                    ph_v[pl.ds(i * D + dc * 16, 16)] = ph
                    pt_v[pl.ds(i * D + dc * 16, 16)] = pt
                    diff = ph + r_b[i, sl] - pt
                    acc = diff * diff if acc is None else acc + diff * diff
                sc_part[pl.ds(i * PAD, 16)] = acc
            sc = plsc.load_gather(sc_part, [grow])
            for l in range(1, 16):
                sc = sc + plsc.load_gather(sc_part, [grow + l])
            sc_v[pl.ds(g * 16, 16)] = sc
            return gcarry

        lax.fori_loop(0, GROUPS, group_body, 0)
        pltpu.sync_copy(ph_v, ph_hbm.at[pl.ds(off * D, CHUNK * D)])
        pltpu.sync_copy(pt_v, pt_hbm.at[pl.ds(off * D, CHUNK * D)])
        pltpu.sync_copy(sc_v, score_hbm.at[pl.ds(off, CHUNK)])

    fire(0, h_a, t_a, r_a, sem_a)

    def pair_body(cc, carry):
        ca = 2 * cc
        cb = 2 * cc + 1
        fire(cb, h_b2, t_b2, r_b2, sem_b)
        drain(ca, h_a, t_a, r_a, sem_a)
        compute(ca, h_a, t_a, r_a)

        @pl.when(cc + 1 < NCHUNK // 2)
        def _():
            fire(ca + 2, h_a, t_a, r_a, sem_a)

        drain(cb, h_b2, t_b2, r_b2, sem_b)
        compute(cb, h_b2, t_b2, r_b2)
        return carry

    lax.fori_loop(0, NCHUNK // 2, pair_body, 0)


def kernel(ratings, triples, ent_emb, rel_emb, ent_proj_emb, rel_proj_emb):
    h = triples[0]
    t = triples[1]
    r = triples[2]
    entc = jnp.concatenate([ent_emb, ent_proj_emb], axis=1)
    relc = jnp.concatenate([rel_emb, rel_proj_emb], axis=1)
    score, ph_flat, pt_flat = _sc_transd(h, t, r, entc, relc)
    proj_h_e = ph_flat.reshape(B, D)
    proj_t_e = pt_flat.reshape(B, D)
    ones = jnp.ones((512, 64), dtype=jnp.float32)
    return (score, proj_h_e, proj_t_e, ones, ones)


# speedup vs baseline: 1.8735x; 1.0000x over previous
import functools

import jax
import jax.numpy as jnp
from jax import lax
from jax.experimental import pallas as pl
from jax.experimental.pallas import tpu as pltpu
from jax.experimental.pallas import tpu_sc as plsc

B = 16384
D = 64
DC = D // 16
NC = 2
NS = 16
NW = NC * NS
PER_W = B // NW
CHUNK = 128
NCHUNK = PER_W // CHUNK
GROUPS = CHUNK // 16
PAD = 17

_MESH = plsc.VectorSubcoreMesh(core_axis_name="c", subcore_axis_name="s")

_BCAST_DNUMS = lax.GatherDimensionNumbers(
    offset_dims=(), collapsed_slice_dims=(0,), start_index_map=(0,))


def _lane_broadcast(vec, j):
    idx = jnp.full((16, 1), j, jnp.int32)
    return lax.gather(vec, idx, _BCAST_DNUMS, (1,),
                      mode=lax.GatherScatterMode.PROMISE_IN_BOUNDS)


@functools.partial(
    pl.kernel,
    mesh=_MESH,
    compiler_params=pltpu.CompilerParams(
        needs_layout_passes=False, use_tc_tiling_on_sc=True),
    out_type=[
        jax.ShapeDtypeStruct((B,), jnp.float32),
        jax.ShapeDtypeStruct((B * D,), jnp.float32),
        jax.ShapeDtypeStruct((B * D,), jnp.float32),
    ],
    scratch_types=[
        pltpu.VMEM((PER_W,), jnp.int32),
        pltpu.VMEM((PER_W,), jnp.int32),
        pltpu.VMEM((PER_W,), jnp.int32),
        pltpu.VMEM((CHUNK, 2 * D), jnp.float32),
        pltpu.VMEM((CHUNK, 2 * D), jnp.float32),
        pltpu.VMEM((CHUNK, 2 * D), jnp.float32),
        pltpu.VMEM((CHUNK, 2 * D), jnp.float32),
        pltpu.VMEM((CHUNK, 2 * D), jnp.float32),
        pltpu.VMEM((CHUNK, 2 * D), jnp.float32),
        pltpu.VMEM((CHUNK * PAD,), jnp.float32),
        pltpu.VMEM((CHUNK * PAD,), jnp.float32),
        pltpu.VMEM((CHUNK * PAD,), jnp.float32),
        pltpu.VMEM((CHUNK * D,), jnp.float32),
        pltpu.VMEM((CHUNK * D,), jnp.float32),
        pltpu.VMEM((CHUNK,), jnp.float32),
        pltpu.SemaphoreType.DMA,
        pltpu.SemaphoreType.DMA,
    ],
)
def _sc_transd(h_hbm, t_hbm, r_hbm, entc_hbm, relc_hbm,
               score_hbm, ph_hbm, pt_hbm,
               h_idx, t_idx, r_idx,
               h_a, t_a, r_a, h_b2, t_b2, r_b2,
               hd_part, td_part, sc_part,
               ph_v, pt_v, sc_v, sem_a, sem_b):
    wid = lax.axis_index("s") * NC + lax.axis_index("c")
    base = wid * PER_W
    iota16 = lax.iota(jnp.int32, 16)
    row17 = iota16 * PAD

    pltpu.sync_copy(h_hbm.at[pl.ds(base, PER_W)], h_idx)
    pltpu.sync_copy(t_hbm.at[pl.ds(base, PER_W)], t_idx)
    pltpu.sync_copy(r_hbm.at[pl.ds(base, PER_W)], r_idx)

    def fire(c, hb, tb, rb, sem):
        sl = pl.ds(c * CHUNK, CHUNK)
        pltpu.async_copy(entc_hbm.at[h_idx.at[sl]], hb, sem)
        pltpu.async_copy(entc_hbm.at[t_idx.at[sl]], tb, sem)
        pltpu.async_copy(relc_hbm.at[r_idx.at[sl]], rb, sem)

    def drain(hb, tb, rb, sem):
        sl = pl.ds(0, CHUNK)
        pltpu.make_async_copy(entc_hbm.at[h_idx.at[sl]], hb, sem).wait()
        pltpu.make_async_copy(entc_hbm.at[t_idx.at[sl]], tb, sem).wait()
        pltpu.make_async_copy(relc_hbm.at[r_idx.at[sl]], rb, sem).wait()

    def compute(c, h_b, t_b, r_b):
        off = base + c * CHUNK

        def dot_body(i, dcarry):
            hd = h_b[i, pl.ds(0, 16)] * h_b[i, pl.ds(D, 16)]
            td = t_b[i, pl.ds(0, 16)] * t_b[i, pl.ds(D, 16)]
            for dc in range(1, DC):
                sl = pl.ds(dc * 16, 16)
                slp = pl.ds(D + dc * 16, 16)
                hd = hd + h_b[i, sl] * h_b[i, slp]
                td = td + t_b[i, sl] * t_b[i, slp]
            hd_part[pl.ds(i * PAD, 16)] = hd
            td_part[pl.ds(i * PAD, 16)] = td
            return dcarry

        lax.fori_loop(0, CHUNK, dot_body, 0)

        def group_body(g, gcarry):
            grow = g * (16 * PAD) + row17
            sh = plsc.load_gather(hd_part, [grow])
            st = plsc.load_gather(td_part, [grow])
            for l in range(1, 16):
                gl = grow + l
                sh = sh + plsc.load_gather(hd_part, [gl])
                st = st + plsc.load_gather(td_part, [gl])
            for j in range(16):
                i = g * 16 + j
                shi = _lane_broadcast(sh, j)
                sti = _lane_broadcast(st, j)
                acc = None
                for dc in range(DC):
                    sl = pl.ds(dc * 16, 16)
                    slp = pl.ds(D + dc * 16, 16)
                    rp = r_b[i, slp]
                    ph = h_b[i, sl] + shi * rp
                    pt = t_b[i, sl] + sti * rp
                    ph_v[pl.ds(i * D + dc * 16, 16)] = ph
                    pt_v[pl.ds(i * D + dc * 16, 16)] = pt
                    diff = ph + r_b[i, sl] - pt
                    acc = diff * diff if acc is None else acc + diff * diff
                sc_part[pl.ds(i * PAD, 16)] = acc
            sc = plsc.load_gather(sc_part, [grow])
            for l in range(1, 16):
                sc = sc + plsc.load_gather(sc_part, [grow + l])
            sc_v[pl.ds(g * 16, 16)] = sc
            return gcarry

        lax.fori_loop(0, GROUPS, group_body, 0)
        pltpu.sync_copy(ph_v, ph_hbm.at[pl.ds(off * D, CHUNK * D)])
        pltpu.sync_copy(pt_v, pt_hbm.at[pl.ds(off * D, CHUNK * D)])
        pltpu.sync_copy(sc_v, score_hbm.at[pl.ds(off, CHUNK)])

    fire(0, h_a, t_a, r_a, sem_a)

    def pair_body(cc, carry):
        ca = 2 * cc
        cb = 2 * cc + 1
        fire(cb, h_b2, t_b2, r_b2, sem_b)
        drain(h_a, t_a, r_a, sem_a)
        compute(ca, h_a, t_a, r_a)

        @pl.when(cc + 1 < NCHUNK // 2)
        def _():
            fire(ca + 2, h_a, t_a, r_a, sem_a)

        drain(h_b2, t_b2, r_b2, sem_b)
        compute(cb, h_b2, t_b2, r_b2)
        return carry

    lax.fori_loop(0, NCHUNK // 2, pair_body, 0)


def kernel(ratings, triples, ent_emb, rel_emb, ent_proj_emb, rel_proj_emb):
    h = triples[0]
    t = triples[1]
    r = triples[2]
    entc = jnp.concatenate([ent_emb, ent_proj_emb], axis=1)
    relc = jnp.concatenate([rel_emb, rel_proj_emb], axis=1)
    score, ph_flat, pt_flat = _sc_transd(h, t, r, entc, relc)
    proj_h_e = ph_flat.reshape(B, D)
    proj_t_e = pt_flat.reshape(B, D)
    ones = jnp.ones((512, 64), dtype=jnp.float32)
    return (score, proj_h_e, proj_t_e, ones, ones)
